# baseline (speedup 1.0000x reference)
import functools

import jax
import jax.numpy as jnp
from jax import lax
from jax.experimental import pallas as pl
from jax.experimental.pallas import tpu as pltpu
from jax.experimental.pallas import tpu_sc as plsc

B = 16384
D = 64
F = 84
NW = 32
BPW = B // NW
H = BPW // 2
U = 4


def _sc_kernel(y_hbm, x_hbm, idx_hbm, t_hbm, table_hbm, out_hbm,
               idx_v, rows_v, out_v, g1s, g2s, s1, s2, s3):
    wid = lax.axis_index("s") * 2 + lax.axis_index("c")
    base = wid * BPW

    pltpu.sync_copy(idx_hbm.at[pl.ds(base, BPW)], idx_v)
    g1 = pltpu.async_copy(table_hbm.at[idx_v.at[pl.ds(0, H)]],
                          rows_v.at[pl.ds(0, H)], g1s)
    g2 = pltpu.async_copy(table_hbm.at[idx_v.at[pl.ds(H, H)]],
                          rows_v.at[pl.ds(H, H)], g2s)

    a = pltpu.async_copy(y_hbm.at[pl.ds(base, BPW)],
                         out_v.at[:, :, pl.ds(0, 8)], s1)
    b = pltpu.async_copy(x_hbm.at[pl.ds(base, BPW)],
                         out_v.at[:, :, pl.ds(8, 8)], s2)
    c = pltpu.async_copy(t_hbm.at[pl.ds(base, BPW)],
                         out_v.at[:, :, pl.ds(80, 4)], s3)

    def reloc(lo):
        def body(i, carry):
            r = lo + i * U
            for u in range(U):
                for k in range(4):
                    out_v[r + u, 0, pl.ds(16 + 16 * k, 16)] = (
                        rows_v[r + u, pl.ds(16 * k, 16)])
            return carry
        lax.fori_loop(0, H // U, body, None)

    g1.wait()
    reloc(0)
    g2.wait()
    reloc(H)

    a.wait(); b.wait(); c.wait()
    pltpu.sync_copy(out_v, out_hbm.at[pl.ds(base, BPW)])


@jax.jit
def _run(y3, x3, idx, t3, table):
    mesh = plsc.VectorSubcoreMesh(core_axis_name="c", subcore_axis_name="s")
    f = functools.partial(
        pl.kernel, mesh=mesh,
        compiler_params=pltpu.CompilerParams(use_tc_tiling_on_sc=False),
        out_type=jax.ShapeDtypeStruct((B, 1, F), jnp.float32),
        scratch_types=[
            pltpu.VMEM((BPW,), jnp.int32),
            pltpu.VMEM((BPW, D), jnp.float32),
            pltpu.VMEM((BPW, 1, F), jnp.float32),
            pltpu.SemaphoreType.DMA,
            pltpu.SemaphoreType.DMA,
            pltpu.SemaphoreType.DMA,
            pltpu.SemaphoreType.DMA,
            pltpu.SemaphoreType.DMA,
        ],
    )(_sc_kernel)
    return f(y3, x3, idx, t3, table)


def kernel(gsp_y_osgb_fourier, gsp_x_osgb_fourier, gsp_id,
           gsp_5_min_time_utc_fourier, emb_table):
    idx = gsp_id.astype(jnp.int32)
    t3 = gsp_5_min_time_utc_fourier[:, None, :]
    return _run(gsp_y_osgb_fourier, gsp_x_osgb_fourier, idx, t3, emb_table)


# speedup vs baseline: 1.9941x; 1.9941x over previous
import functools

import jax
import jax.numpy as jnp
from jax import lax
from jax.experimental import pallas as pl
from jax.experimental.pallas import tpu as pltpu
from jax.experimental.pallas import tpu_sc as plsc

B = 16384
D = 64
F = 84
NW = 32
BPW = B // NW
H = BPW // 2


def _sc_kernel(yxt_hbm, idx_hbm, table_hbm, out_hbm,
               idx_v, rows_v, yxt_v, g1s, g2s, fs, w1s, w2s, w3s, w4s):
    wid = lax.axis_index("s") * 2 + lax.axis_index("c")
    base = wid * BPW

    pltpu.sync_copy(idx_hbm.at[pl.ds(base, BPW)], idx_v)
    g1 = pltpu.async_copy(table_hbm.at[idx_v.at[pl.ds(0, H)]],
                          rows_v.at[pl.ds(0, H)], g1s)
    g2 = pltpu.async_copy(table_hbm.at[idx_v.at[pl.ds(H, H)]],
                          rows_v.at[pl.ds(H, H)], g2s)

    f = pltpu.async_copy(yxt_hbm.at[pl.ds(base, BPW)], yxt_v, fs)
    f.wait()
    w1 = pltpu.async_copy(yxt_v.at[:, pl.ds(0, 16)],
                          out_hbm.at[pl.ds(base, BPW), pl.ds(0, 16)], w1s)
    w2 = pltpu.async_copy(yxt_v.at[:, pl.ds(16, 4)],
                          out_hbm.at[pl.ds(base, BPW), pl.ds(80, 4)], w2s)

    g1.wait()
    w3 = pltpu.async_copy(rows_v.at[pl.ds(0, H)],
                          out_hbm.at[pl.ds(base, H), pl.ds(16, D)], w3s)
    g2.wait()
    w4 = pltpu.async_copy(rows_v.at[pl.ds(H, H)],
                          out_hbm.at[pl.ds(base + H, H), pl.ds(16, D)], w4s)

    w1.wait(); w2.wait(); w3.wait(); w4.wait()


@jax.jit
def _run(yxt, idx, table):
    mesh = plsc.VectorSubcoreMesh(core_axis_name="c", subcore_axis_name="s")
    f = functools.partial(
        pl.kernel, mesh=mesh,
        compiler_params=pltpu.CompilerParams(use_tc_tiling_on_sc=False),
        out_type=jax.ShapeDtypeStruct((B, F), jnp.float32),
        scratch_types=[
            pltpu.VMEM((BPW,), jnp.int32),
            pltpu.VMEM((BPW, D), jnp.float32),
            pltpu.VMEM((BPW, 20), jnp.float32),
            pltpu.SemaphoreType.DMA,
            pltpu.SemaphoreType.DMA,
            pltpu.SemaphoreType.DMA,
            pltpu.SemaphoreType.DMA,
            pltpu.SemaphoreType.DMA,
            pltpu.SemaphoreType.DMA,
            pltpu.SemaphoreType.DMA,
        ],
    )(_sc_kernel)
    return f(yxt, idx, table)


def kernel(gsp_y_osgb_fourier, gsp_x_osgb_fourier, gsp_id,
           gsp_5_min_time_utc_fourier, emb_table):
    yxt = jnp.concatenate(
        [gsp_y_osgb_fourier[:, 0], gsp_x_osgb_fourier[:, 0],
         gsp_5_min_time_utc_fourier], axis=1)
    idx = gsp_id.astype(jnp.int32)
    out = _run(yxt, idx, emb_table)
    return out[:, None, :]
